# baseline (speedup 1.0000x reference)
import jax
import jax.numpy as jnp
from jax import lax
from jax.experimental import pallas as pl
from jax.experimental.pallas import tpu as pltpu
from jax.experimental.pallas import tpu_sc as plsc

DAT_NUM = 1000000
DEL_NUM = 32
BATCH = 16384

_NC = 2
_NS = 16
_NW = _NC * _NS
_BPW = BATCH // _NW
_CHUNK = 128
_NCHUNK = _BPW // _CHUNK


def _body(idx_hbm, table_hbm, out_hbm, idx_v, rows_v, stage_v, sem):
    wid = lax.axis_index("s") * _NC + lax.axis_index("c")
    base = wid * _BPW

    pltpu.sync_copy(idx_hbm.at[pl.ds(wid * _NCHUNK, _NCHUNK)], idx_v)

    copies = []
    for j in range(_NCHUNK):
        copies.append(
            pltpu.async_copy(
                table_hbm.at[idx_v.at[j]],
                rows_v.at[pl.ds(j * _CHUNK, _CHUNK)],
                sem,
            )
        )
    for c in copies:
        c.wait()

    c16 = lax.iota(jnp.int32, 16)

    def bc16(s):
        return jnp.broadcast_to(s, (16,)).astype(jnp.int32)

    def grp(g, carry):
        row16 = g * 16 + c16
        for c in range(DEL_NUM):
            v = plsc.load_gather(rows_v, [row16, bc16(c)])
            s = 1.0 / (1.0 + jnp.exp(-v))
            plsc.store_scatter(stage_v, [bc16(c), row16], s)
        return carry

    lax.fori_loop(0, _BPW // 16, grp, 0)

    copies = []
    for c in range(DEL_NUM):
        copies.append(
            pltpu.async_copy(
                stage_v.at[c],
                out_hbm.at[pl.ds(c * BATCH + base, _BPW)],
                sem,
            )
        )
    for cp in copies:
        cp.wait()


@jax.jit
def _sc_lookup_sigmoid(idx, table):
    mesh = plsc.VectorSubcoreMesh(core_axis_name="c", subcore_axis_name="s")
    k = pl.kernel(
        _body,
        out_type=jax.ShapeDtypeStruct((DEL_NUM * BATCH,), jnp.float32),
        mesh=mesh,
        scratch_types=[
            pltpu.VMEM((_NCHUNK, _CHUNK), jnp.int32),
            pltpu.VMEM((_BPW, DEL_NUM), jnp.float32),
            pltpu.VMEM((DEL_NUM, _BPW), jnp.float32),
            pltpu.SemaphoreType.DMA,
        ],
        compiler_params=pltpu.CompilerParams(
            needs_layout_passes=False, use_tc_tiling_on_sc=False
        ),
    )
    return k(idx.reshape(_NW * _NCHUNK, _CHUNK), table)


def kernel(idx, table):
    flat = _sc_lookup_sigmoid(idx.astype(jnp.int32), table)
    return flat.reshape(DEL_NUM, BATCH).T[:, :, None]
